# Initial kernel scaffold; baseline (speedup 1.0000x reference)
import jax
import jax.numpy as jnp
from jax.experimental import pallas as pl


def kernel(center, context, negative, in_embed, out_embed):
    raise NotImplementedError("write your pallas kernel here")


# speedup vs baseline: 4.0810x; 4.0810x over previous
import jax
import jax.numpy as jnp
from jax import lax
from jax.experimental import pallas as pl
from jax.experimental.pallas import tpu as pltpu
from jax.experimental.pallas import tpu_sc as plsc

VOCAB = 1000000
EMBED = 64
BATCH = 16384
K = 21
NC = 2
NS = 16
NW = NC * NS
EPW = BATCH // NW
CHUNK = 64
NCHUNK = EPW // CHUNK
L = 16


def _sc_body(center_hbm, oidx_hbm, in_embed_hbm, out_embed_hbm, scores_hbm,
             cidx_v, oidx_v, crows_v, orows_v, scores_v, sem_i, sem_c, sem_o):
    cid = lax.axis_index("c")
    sid = lax.axis_index("s")
    wid = sid * NC + cid

    lane = lax.iota(jnp.int32, L)

    def chunk_body(i, carry):
        base = wid * EPW + i * CHUNK
        i1 = pltpu.async_copy(center_hbm.at[pl.ds(base, CHUNK)], cidx_v, sem_i)
        i2 = pltpu.async_copy(oidx_hbm.at[:, pl.ds(base, CHUNK)], oidx_v,
                              sem_i)
        i1.wait()
        i2.wait()
        gdmas = [pltpu.async_copy(in_embed_hbm.at[cidx_v], crows_v, sem_c)]
        for k in range(K):
            gdmas.append(
                pltpu.async_copy(out_embed_hbm.at[oidx_v.at[k]],
                                 orows_v.at[k], sem_o))
        for d in gdmas:
            d.wait()

        for g in range(CHUNK // L):
            e_idx = lane + (g * L)
            def d_body(d, accs):
                dv = jnp.zeros((L,), jnp.int32) + d
                cv = plsc.load_gather(crows_v, [e_idx, dv])
                return tuple(
                    accs[k] + plsc.load_gather(orows_v.at[k], [e_idx, dv]) * cv
                    for k in range(K)
                )
            accs = lax.fori_loop(
                0, EMBED, d_body,
                tuple(jnp.zeros((L,), jnp.float32) for _ in range(K)))
            for k in range(K):
                scores_v[k, pl.ds(g * L, L)] = accs[k]

        pltpu.sync_copy(scores_v, scores_hbm.at[:, pl.ds(base, CHUNK)])
        return carry

    lax.fori_loop(0, NCHUNK, chunk_body, 0)


def _sc_scores(center, oidx, in_embed, out_embed):
    mesh = plsc.VectorSubcoreMesh(core_axis_name="c", subcore_axis_name="s")
    return pl.kernel(
        _sc_body,
        out_type=jax.ShapeDtypeStruct((K, BATCH), jnp.float32),
        mesh=mesh,
        compiler_params=pltpu.CompilerParams(
            use_tc_tiling_on_sc=False,
            needs_layout_passes=False,
        ),
        scratch_types=[
            pltpu.VMEM((CHUNK,), jnp.int32),
            pltpu.VMEM((K, CHUNK), jnp.int32),
            pltpu.VMEM((CHUNK, EMBED), jnp.float32),
            pltpu.VMEM((K, CHUNK, EMBED), jnp.float32),
            pltpu.VMEM((K, CHUNK), jnp.float32),
            pltpu.SemaphoreType.DMA,
            pltpu.SemaphoreType.DMA,
            pltpu.SemaphoreType.DMA,
        ],
    )(center, oidx, in_embed, out_embed)


def _loss_body(s_ref, o_ref):
    s = s_ref[...]
    pos = s[0:1, :]
    neg = s[1:, :]
    total = (jnp.sum(jnp.log(jax.nn.sigmoid(pos))) +
             jnp.sum(jnp.log(jax.nn.sigmoid(-neg))))
    o_ref[...] = jnp.reshape(-total / BATCH, (1, 1))


def _loss(scores):
    out = pl.pallas_call(
        _loss_body,
        out_shape=jax.ShapeDtypeStruct((1, 1), jnp.float32),
    )(scores)
    return out[0, 0]


@jax.jit
def kernel(center, context, negative, in_embed, out_embed):
    oidx = jnp.concatenate([context[None, :], negative.T], axis=0)
    scores = _sc_scores(center, oidx, in_embed, out_embed)
    return _loss(scores)
